# Initial kernel scaffold; baseline (speedup 1.0000x reference)
import jax
import jax.numpy as jnp
from jax.experimental import pallas as pl


def kernel(x, tags, pe):
    raise NotImplementedError("write your pallas kernel here")


# speedup vs baseline: 1.2627x; 1.2627x over previous
import jax
import jax.numpy as jnp
from jax.experimental import pallas as pl
from jax.experimental.pallas import tpu as pltpu


def _body(tags_ref, x_ref, pe_ref, o_ref):
    i = pl.program_id(0)
    tag = tags_ref[i]
    o_ref[0] = x_ref[0] + pe_ref[tag]


def kernel(x, tags, pe):
    B, T, S, D = x.shape
    n = B * T
    x2 = x.reshape(n, S, D)
    tags_i = tags.reshape(-1).astype(jnp.int32)
    grid_spec = pltpu.PrefetchScalarGridSpec(
        num_scalar_prefetch=1,
        grid=(n,),
        in_specs=[
            pl.BlockSpec((1, S, D), lambda i, tags_r: (i, 0, 0)),
            pl.BlockSpec((pe.shape[0], S, D), lambda i, tags_r: (0, 0, 0)),
        ],
        out_specs=pl.BlockSpec((1, S, D), lambda i, tags_r: (i, 0, 0)),
    )
    out = pl.pallas_call(
        _body,
        grid_spec=grid_spec,
        out_shape=jax.ShapeDtypeStruct((n, S, D), x.dtype),
    )(tags_i, x2, pe)
    return out.reshape(B, T, S, D)


# speedup vs baseline: 1.9356x; 1.5330x over previous
import jax
import jax.numpy as jnp
from jax.experimental import pallas as pl
from jax.experimental.pallas import tpu as pltpu


_ROWS = 8


def _body(tags_ref, x_ref, pe_ref, o_ref):
    i = pl.program_id(0)
    for r in range(_ROWS):
        tag = tags_ref[i * _ROWS + r]
        o_ref[r] = x_ref[r] + pe_ref[tag]


def kernel(x, tags, pe):
    B, T, S, D = x.shape
    n = B * T
    x2 = x.reshape(n, S, D)
    tags_i = tags.reshape(-1).astype(jnp.int32)
    grid_spec = pltpu.PrefetchScalarGridSpec(
        num_scalar_prefetch=1,
        grid=(n // _ROWS,),
        in_specs=[
            pl.BlockSpec((_ROWS, S, D), lambda i, tags_r: (i, 0, 0)),
            pl.BlockSpec((pe.shape[0], S, D), lambda i, tags_r: (0, 0, 0)),
        ],
        out_specs=pl.BlockSpec((_ROWS, S, D), lambda i, tags_r: (i, 0, 0)),
    )
    out = pl.pallas_call(
        _body,
        grid_spec=grid_spec,
        out_shape=jax.ShapeDtypeStruct((n, S, D), x.dtype),
    )(tags_i, x2, pe)
    return out.reshape(B, T, S, D)
